# Initial kernel scaffold; baseline (speedup 1.0000x reference)
import functools

import jax
import jax.numpy as jnp
from jax import lax
from jax.experimental import pallas as pl
from jax.experimental.pallas import tpu as pltpu
from jax.experimental.pallas import tpu_sc as plsc

N_NODES = 10000
D_IN = 256
D_OUT = 256
N_EDGES = 160000

HALF = D_OUT // 2
N_TILES = 16
E_TILE = N_EDGES // N_TILES
K = 80
N_CHUNKS = E_TILE // K
ROWS_TILE = N_NODES // N_TILES


def _mm_body(x_ref, w_ref, o_ref):
    o_ref[...] = jnp.dot(x_ref[...], w_ref[...],
                         preferred_element_type=jnp.float32)


def _matmul_halves(x, W):
    BN = 1000
    return pl.pallas_call(
        _mm_body,
        grid=(2, N_NODES // BN),
        in_specs=[
            pl.BlockSpec((BN, D_IN), lambda c, i: (i, 0)),
            pl.BlockSpec((D_IN, HALF), lambda c, i: (0, c)),
        ],
        out_specs=pl.BlockSpec((BN, HALF), lambda c, i: (c * (N_NODES // BN) + i, 0)),
        out_shape=jax.ShapeDtypeStruct((2 * N_NODES, HALF), jnp.float32),
    )(x, W)


def _sc_aggregate(h2, src2, dst_r, zeros):
    mesh = plsc.VectorSubcoreMesh(core_axis_name="c", subcore_axis_name="s")

    @functools.partial(
        pl.kernel,
        mesh=mesh,
        out_type=jax.ShapeDtypeStruct((2, N_NODES, HALF), jnp.float32),
        scratch_types=[
            pltpu.VMEM((N_CHUNKS, K), jnp.int32),
            pltpu.VMEM((N_CHUNKS, K), jnp.int32),
            pltpu.VMEM((K, HALF), jnp.float32),
            pltpu.VMEM_SHARED((N_NODES, HALF), jnp.float32),
            pltpu.SemaphoreType.DMA,
        ],
    )
    def agg(h2_hbm, src2_hbm, dst_hbm, zeros_hbm, out_hbm,
            src_v, dst_v, buf, acc, sem):
        c = lax.axis_index("c")
        s = lax.axis_index("s")

        pltpu.sync_copy(zeros_hbm.at[pl.ds(s * ROWS_TILE, ROWS_TILE)],
                        acc.at[pl.ds(s * ROWS_TILE, ROWS_TILE)])

        pltpu.sync_copy(src2_hbm.at[c, pl.ds(s * N_CHUNKS, N_CHUNKS)], src_v)
        pltpu.sync_copy(dst_hbm.at[pl.ds(s * N_CHUNKS, N_CHUNKS)], dst_v)

        plsc.subcore_barrier()

        def body(j, carry):
            pltpu.async_copy(h2_hbm.at[src_v.at[j]], buf, sem).wait()
            pltpu.sync_copy(buf, acc.at[dst_v.at[j]], add=True)
            return carry

        lax.fori_loop(0, N_CHUNKS, body, 0)

        plsc.subcore_barrier()

        pltpu.sync_copy(acc.at[pl.ds(s * ROWS_TILE, ROWS_TILE)],
                        out_hbm.at[c, pl.ds(s * ROWS_TILE, ROWS_TILE)])

    return agg(h2, src2, dst_r, zeros)


def kernel(x, edge_index, W):
    src = edge_index[0].astype(jnp.int32)
    dst = edge_index[1].astype(jnp.int32)

    h2 = _matmul_halves(x, W)

    src2 = jnp.stack([src, src + N_NODES]).reshape(2, N_TILES * N_CHUNKS, K)
    dst_r = dst.reshape(N_TILES * N_CHUNKS, K)
    zeros = jnp.zeros((N_NODES, HALF), jnp.float32)

    out2 = _sc_aggregate(h2, src2, dst_r, zeros)
    return jnp.concatenate([out2[0], out2[1]], axis=1)


# speedup vs baseline: 4.9700x; 4.9700x over previous
import functools

import jax
import jax.numpy as jnp
from jax import lax
from jax.experimental import pallas as pl
from jax.experimental.pallas import tpu as pltpu
from jax.experimental.pallas import tpu_sc as plsc

N_NODES = 10000
D_IN = 256
D_OUT = 256
N_EDGES = 160000

HALF = D_OUT // 2
N_TILES = 16
E_TILE = N_EDGES // N_TILES
K = 80
N_CHUNKS = E_TILE // K
ACC_ROWS = 10240
ROWS_TILE = ACC_ROWS // N_TILES


def _mm_body(x_ref, w_ref, o_ref):
    o_ref[...] = jnp.dot(x_ref[...], w_ref[...],
                         preferred_element_type=jnp.float32)


def _matmul_halves(x, W):
    BN = 1000
    return pl.pallas_call(
        _mm_body,
        grid=(2, N_NODES // BN),
        in_specs=[
            pl.BlockSpec((BN, D_IN), lambda c, i: (i, 0)),
            pl.BlockSpec((D_IN, HALF), lambda c, i: (0, c)),
        ],
        out_specs=pl.BlockSpec((BN, HALF), lambda c, i: (c * (N_NODES // BN) + i, 0)),
        out_shape=jax.ShapeDtypeStruct((2 * N_NODES, HALF), jnp.float32),
    )(x, W)


def _sc_aggregate(h2, src2, dst_r, zeros):
    mesh = plsc.VectorSubcoreMesh(core_axis_name="c", subcore_axis_name="s")

    @functools.partial(
        pl.kernel,
        mesh=mesh,
        out_type=jax.ShapeDtypeStruct((2, ACC_ROWS, HALF), jnp.float32),
        scratch_types=[
            pltpu.VMEM((N_CHUNKS, K), jnp.int32),
            pltpu.VMEM((N_CHUNKS, K), jnp.int32),
            pltpu.VMEM((K, HALF), jnp.float32),
            pltpu.VMEM_SHARED((ACC_ROWS, HALF), jnp.float32),
            pltpu.SemaphoreType.DMA,
        ],
    )
    def agg(h2_hbm, src2_hbm, dst_hbm, zeros_hbm, out_hbm,
            src_v, dst_v, buf, acc, sem):
        c = lax.axis_index("c")
        s = lax.axis_index("s")

        pltpu.sync_copy(zeros_hbm.at[pl.ds(s * ROWS_TILE, ROWS_TILE)],
                        acc.at[pl.ds(s * ROWS_TILE, ROWS_TILE)])

        pltpu.sync_copy(src2_hbm.at[c, s], src_v)
        pltpu.sync_copy(dst_hbm.at[s], dst_v)

        plsc.subcore_barrier()

        def body(j, carry):
            pltpu.async_copy(h2_hbm.at[src_v.at[j]], buf, sem).wait()
            pltpu.sync_copy(buf, acc.at[dst_v.at[j]], add=True)
            return carry

        lax.fori_loop(0, N_CHUNKS, body, 0)

        plsc.subcore_barrier()

        pltpu.sync_copy(acc.at[pl.ds(s * ROWS_TILE, ROWS_TILE)],
                        out_hbm.at[c, pl.ds(s * ROWS_TILE, ROWS_TILE)])

    return agg(h2, src2, dst_r, zeros)


def kernel(x, edge_index, W):
    src = edge_index[0].astype(jnp.int32)
    dst = edge_index[1].astype(jnp.int32)

    h2 = _matmul_halves(x, W)

    src2 = jnp.stack([src, src + N_NODES]).reshape(2, N_TILES, N_CHUNKS, K)
    dst_r = dst.reshape(N_TILES, N_CHUNKS, K)
    zeros = jnp.zeros((ACC_ROWS, HALF), jnp.float32)

    out2 = _sc_aggregate(h2, src2, dst_r, zeros)
    return jnp.concatenate([out2[0, :N_NODES], out2[1, :N_NODES]], axis=1)
